# baseline (speedup 1.0000x reference)
import functools

import jax
import jax.numpy as jnp
from jax import lax
from jax.experimental import pallas as pl
from jax.experimental.pallas import tpu as pltpu
from jax.experimental.pallas import tpu_sc as plsc

EPS = 1e-12
H = 128
LANES = 16
NJ = H // LANES
CHUNK = 128


def _make_sc_call(n_rows, seq_len):
  nc, ns = 2, 16
  nw = nc * ns
  assert n_rows % (nw * CHUNK) == 0
  rows_per_w = n_rows // nw
  assert rows_per_w % seq_len == 0
  n_chunks = rows_per_w // CHUNK
  assert n_chunks >= 2 and n_chunks % 2 == 0

  mesh = plsc.VectorSubcoreMesh(
      core_axis_name="c", subcore_axis_name="s", num_cores=nc, num_subcores=ns)

  @functools.partial(
      pl.kernel,
      out_type=jax.ShapeDtypeStruct((n_rows, H), jnp.float32),
      mesh=mesh,
      compiler_params=pltpu.CompilerParams(needs_layout_passes=False),
      scratch_types=[
          pltpu.VMEM((CHUNK,), jnp.int32),
          pltpu.VMEM((CHUNK,), jnp.int32),
          pltpu.VMEM((CHUNK, H), jnp.float32),
          pltpu.VMEM((CHUNK, H), jnp.float32),
          pltpu.VMEM((CHUNK, H), jnp.float32),
          pltpu.VMEM((CHUNK, H), jnp.float32),
          pltpu.VMEM((seq_len, H), jnp.float32),
          pltpu.VMEM((1, H), jnp.float32),
          pltpu.VMEM((H,), jnp.float32),
          pltpu.VMEM((H,), jnp.float32),
          pltpu.SemaphoreType.DMA,
          pltpu.SemaphoreType.DMA,
          pltpu.SemaphoreType.DMA,
          pltpu.SemaphoreType.DMA,
          pltpu.SemaphoreType.DMA,
          pltpu.SemaphoreType.DMA,
      ],
  )
  def call(ids_hbm, word_hbm, pos_hbm, tok_hbm, gam_hbm, bet_hbm, out_hbm,
           idx0, idx1, buf0, buf1, obuf0, obuf1, addv, tokv, gamv, betv,
           sem0, sem1, osem0, osem1, isem0, isem1):
    wid = lax.axis_index("s") * nc + lax.axis_index("c")
    base = wid * rows_per_w

    pltpu.sync_copy(pos_hbm.at[pl.ds(0, seq_len)], addv)
    pltpu.sync_copy(tok_hbm.at[pl.ds(0, 1)], tokv)
    pltpu.sync_copy(gam_hbm, gamv)
    pltpu.sync_copy(bet_hbm, betv)

    def add_tok(i, _):
      for j in range(NJ):
        sl = pl.ds(j * LANES, LANES)
        addv[i, sl] = addv[i, sl] + tokv[0, sl]
      return 0
    lax.fori_loop(0, seq_len, add_tok, 0)

    g_regs = [gamv[pl.ds(j * LANES, LANES)] for j in range(NJ)]
    b_regs = [betv[pl.ds(j * LANES, LANES)] for j in range(NJ)]

    def compute_ln(buf, obuf, chunk_id):
      row0 = chunk_id * CHUNK

      @plsc.parallel_loop(0, CHUNK, unroll=4)
      def row(r):
        p = lax.rem(row0 + r, seq_len)
        t = []
        for j in range(NJ):
          sl = pl.ds(j * LANES, LANES)
          t.append(buf[r, sl] + addv[p, sl])
        sp = [t[2 * j] + t[2 * j + 1] for j in range(4)]
        s = (sp[0] + sp[1]) + (sp[2] + sp[3])
        m = [tj * tj for tj in t]
        qp = [m[2 * j] + m[2 * j + 1] for j in range(4)]
        q = (qp[0] + qp[1]) + (qp[2] + qp[3])
        mean = jnp.sum(s) * jnp.float32(1.0 / H)
        msq = jnp.sum(q) * jnp.float32(1.0 / H)
        var = msq - mean * mean
        x = var + jnp.float32(EPS)
        i32 = lax.bitcast_convert_type(x, jnp.int32)
        i32 = jnp.int32(0x5F3759DF) - lax.shift_right_logical(i32, 1)
        y = lax.bitcast_convert_type(i32, jnp.float32)
        for _ in range(1):
          y = y * (jnp.float32(1.5) - jnp.float32(0.5) * x * y * y)
        c = mean * y
        for j in range(NJ):
          obuf[r, pl.ds(j * LANES, LANES)] = (
              (t[j] * y - c) * g_regs[j] + b_regs[j])

    def idx_src(g):
      return ids_hbm.at[pl.ds(base + g * CHUNK, CHUNK)]

    def out_dst(g):
      return out_hbm.at[pl.ds(base + g * CHUNK, CHUNK)]

    pltpu.async_copy(idx_src(0), idx0, isem0)
    pltpu.async_copy(idx_src(1), idx1, isem1)
    pltpu.make_async_copy(idx_src(0), idx0, isem0).wait()
    pltpu.async_copy(word_hbm.at[idx0], buf0, sem0)

    def body(t, _):
      g0 = 2 * t
      g1 = g0 + 1
      not_last = t < n_chunks // 2 - 1
      not_first = t > 0

      pltpu.make_async_copy(word_hbm.at[idx0], buf0, sem0).wait()

      @pl.when(not_last)
      def _():
        pltpu.async_copy(idx_src(g0 + 2), idx0, isem0)

      pltpu.make_async_copy(idx_src(g1), idx1, isem1).wait()
      pltpu.async_copy(word_hbm.at[idx1], buf1, sem1)

      @pl.when(not_first)
      def _():
        pltpu.make_async_copy(obuf0, out_dst(g0 - 2), osem0).wait()

      compute_ln(buf0, obuf0, g0)
      pltpu.async_copy(obuf0, out_dst(g0), osem0)

      pltpu.make_async_copy(word_hbm.at[idx1], buf1, sem1).wait()

      @pl.when(not_last)
      def _():
        pltpu.async_copy(idx_src(g1 + 2), idx1, isem1)
        pltpu.make_async_copy(idx_src(g0 + 2), idx0, isem0).wait()
        pltpu.async_copy(word_hbm.at[idx0], buf0, sem0)

      @pl.when(not_first)
      def _():
        pltpu.make_async_copy(obuf1, out_dst(g1 - 2), osem1).wait()

      compute_ln(buf1, obuf1, g1)
      pltpu.async_copy(obuf1, out_dst(g1), osem1)
      return 0

    lax.fori_loop(0, n_chunks // 2, body, 0)

    pltpu.make_async_copy(obuf0, out_dst(n_chunks - 2), osem0).wait()
    pltpu.make_async_copy(obuf1, out_dst(n_chunks - 1), osem1).wait()

  return call


def kernel(input_ids, entity_ids, word_emb, pos_emb, tok_emb, ent_emb,
           ln_gamma, ln_beta):
  del entity_ids, ent_emb
  bsz, seq_len = input_ids.shape
  n_rows = bsz * seq_len
  ids = input_ids.reshape(n_rows).astype(jnp.int32)
  call = _make_sc_call(n_rows, seq_len)
  out = call(ids, word_emb, pos_emb, tok_emb, ln_gamma, ln_beta)
  return out.reshape(bsz, seq_len, H)


# speedup vs baseline: 1.1648x; 1.1648x over previous
import functools

import jax
import jax.numpy as jnp
from jax import lax
from jax.experimental import pallas as pl
from jax.experimental.pallas import tpu as pltpu
from jax.experimental.pallas import tpu_sc as plsc

EPS = 1e-12
H = 128
LANES = 16
NJ = H // LANES
CHUNK = 128


def _make_sc_call(n_rows, seq_len):
  nc, ns = 2, 16
  nw = nc * ns
  assert n_rows % (nw * CHUNK) == 0
  rows_per_w = n_rows // nw
  assert rows_per_w % seq_len == 0
  n_chunks = rows_per_w // CHUNK
  assert n_chunks >= 2 and n_chunks % 2 == 0

  mesh = plsc.VectorSubcoreMesh(
      core_axis_name="c", subcore_axis_name="s", num_cores=nc, num_subcores=ns)

  @functools.partial(
      pl.kernel,
      out_type=jax.ShapeDtypeStruct((n_rows, H), jnp.float32),
      mesh=mesh,
      compiler_params=pltpu.CompilerParams(needs_layout_passes=False),
      scratch_types=[
          pltpu.VMEM((CHUNK,), jnp.int32),
          pltpu.VMEM((CHUNK,), jnp.int32),
          pltpu.VMEM((CHUNK, H), jnp.float32),
          pltpu.VMEM((CHUNK, H), jnp.float32),
          pltpu.VMEM((CHUNK, H), jnp.float32),
          pltpu.VMEM((CHUNK, H), jnp.float32),
          pltpu.VMEM((seq_len, H), jnp.float32),
          pltpu.VMEM((1, H), jnp.float32),
          pltpu.VMEM((H,), jnp.float32),
          pltpu.VMEM((H,), jnp.float32),
          pltpu.SemaphoreType.DMA,
          pltpu.SemaphoreType.DMA,
          pltpu.SemaphoreType.DMA,
          pltpu.SemaphoreType.DMA,
          pltpu.SemaphoreType.DMA,
          pltpu.SemaphoreType.DMA,
      ],
  )
  def call(ids_hbm, word_hbm, pos_hbm, tok_hbm, gam_hbm, bet_hbm, out_hbm,
           idx0, idx1, buf0, buf1, obuf0, obuf1, addv, tokv, gamv, betv,
           sem0, sem1, osem0, osem1, isem0, isem1):
    wid = lax.axis_index("s") * nc + lax.axis_index("c")
    base = wid * rows_per_w

    pltpu.sync_copy(pos_hbm.at[pl.ds(0, seq_len)], addv)
    pltpu.sync_copy(tok_hbm.at[pl.ds(0, 1)], tokv)
    pltpu.sync_copy(gam_hbm, gamv)
    pltpu.sync_copy(bet_hbm, betv)

    def add_tok(i, _):
      for j in range(NJ):
        sl = pl.ds(j * LANES, LANES)
        addv[i, sl] = addv[i, sl] + tokv[0, sl]
      return 0
    lax.fori_loop(0, seq_len, add_tok, 0)

    g_regs = [gamv[pl.ds(j * LANES, LANES)] for j in range(NJ)]
    b_regs = [betv[pl.ds(j * LANES, LANES)] for j in range(NJ)]

    def compute_ln(buf, obuf, chunk_id):
      row0 = chunk_id * CHUNK

      @plsc.parallel_loop(0, CHUNK, unroll=4)
      def row(r):
        p = lax.rem(row0 + r, seq_len)
        t = []
        for j in range(NJ):
          sl = pl.ds(j * LANES, LANES)
          t.append(buf[r, sl] + addv[p, sl])
        s = t[0] + t[1]
        q = t[0] * t[0] + t[1] * t[1]
        for j in range(2, NJ):
          s = s + t[j]
          q = q + t[j] * t[j]
        mean = jnp.sum(s) * jnp.float32(1.0 / H)
        msq = jnp.sum(q) * jnp.float32(1.0 / H)
        var = msq - mean * mean
        x = var + jnp.float32(EPS)
        i32 = lax.bitcast_convert_type(x, jnp.int32)
        i32 = jnp.int32(0x5F3759DF) - lax.shift_right_logical(i32, 1)
        y = lax.bitcast_convert_type(i32, jnp.float32)
        for _ in range(1):
          y = y * (jnp.float32(1.5) - jnp.float32(0.5) * x * y * y)
        c = mean * y
        for j in range(NJ):
          obuf[r, pl.ds(j * LANES, LANES)] = (
              (t[j] * y - c) * g_regs[j] + b_regs[j])

    def idx_src(g):
      return ids_hbm.at[pl.ds(base + g * CHUNK, CHUNK)]

    def out_dst(g):
      return out_hbm.at[pl.ds(base + g * CHUNK, CHUNK)]

    pltpu.async_copy(idx_src(0), idx0, isem0)
    pltpu.async_copy(idx_src(1), idx1, isem1)
    pltpu.make_async_copy(idx_src(0), idx0, isem0).wait()
    pltpu.async_copy(word_hbm.at[idx0], buf0, sem0)

    def body(t, _):
      g0 = 2 * t
      g1 = g0 + 1
      not_last = t < n_chunks // 2 - 1
      not_first = t > 0

      pltpu.make_async_copy(word_hbm.at[idx0], buf0, sem0).wait()

      @pl.when(not_last)
      def _():
        pltpu.async_copy(idx_src(g0 + 2), idx0, isem0)

      pltpu.make_async_copy(idx_src(g1), idx1, isem1).wait()
      pltpu.async_copy(word_hbm.at[idx1], buf1, sem1)

      @pl.when(not_first)
      def _():
        pltpu.make_async_copy(obuf0, out_dst(g0 - 2), osem0).wait()

      compute_ln(buf0, obuf0, g0)
      pltpu.async_copy(obuf0, out_dst(g0), osem0)

      pltpu.make_async_copy(word_hbm.at[idx1], buf1, sem1).wait()

      @pl.when(not_last)
      def _():
        pltpu.async_copy(idx_src(g1 + 2), idx1, isem1)
        pltpu.make_async_copy(idx_src(g0 + 2), idx0, isem0).wait()
        pltpu.async_copy(word_hbm.at[idx0], buf0, sem0)

      @pl.when(not_first)
      def _():
        pltpu.make_async_copy(obuf1, out_dst(g1 - 2), osem1).wait()

      compute_ln(buf1, obuf1, g1)
      pltpu.async_copy(obuf1, out_dst(g1), osem1)
      return 0

    lax.fori_loop(0, n_chunks // 2, body, 0)

    pltpu.make_async_copy(obuf0, out_dst(n_chunks - 2), osem0).wait()
    pltpu.make_async_copy(obuf1, out_dst(n_chunks - 1), osem1).wait()

  return call


def kernel(input_ids, entity_ids, word_emb, pos_emb, tok_emb, ent_emb,
           ln_gamma, ln_beta):
  del entity_ids, ent_emb
  bsz, seq_len = input_ids.shape
  n_rows = bsz * seq_len
  ids = input_ids.reshape(n_rows).astype(jnp.int32)
  call = _make_sc_call(n_rows, seq_len)
  out = call(ids, word_emb, pos_emb, tok_emb, ln_gamma, ln_beta)
  return out.reshape(bsz, seq_len, H)


# speedup vs baseline: 1.1844x; 1.0168x over previous
import functools

import jax
import jax.numpy as jnp
from jax import lax
from jax.experimental import pallas as pl
from jax.experimental.pallas import tpu as pltpu
from jax.experimental.pallas import tpu_sc as plsc

EPS = 1e-12
H = 128
LANES = 16
NJ = H // LANES
CHUNK = 128


def _make_sc_call(n_rows, seq_len):
  nc, ns = 2, 16
  nw = nc * ns
  assert n_rows % (nw * CHUNK) == 0
  rows_per_w = n_rows // nw
  assert rows_per_w % seq_len == 0
  n_chunks = rows_per_w // CHUNK
  assert n_chunks >= 2 and n_chunks % 2 == 0

  mesh = plsc.VectorSubcoreMesh(
      core_axis_name="c", subcore_axis_name="s", num_cores=nc, num_subcores=ns)

  @functools.partial(
      pl.kernel,
      out_type=jax.ShapeDtypeStruct((n_rows, H), jnp.float32),
      mesh=mesh,
      compiler_params=pltpu.CompilerParams(needs_layout_passes=False),
      scratch_types=[
          pltpu.VMEM((CHUNK,), jnp.int32),
          pltpu.VMEM((CHUNK,), jnp.int32),
          pltpu.VMEM((CHUNK, H), jnp.float32),
          pltpu.VMEM((CHUNK, H), jnp.float32),
          pltpu.VMEM((CHUNK, H), jnp.float32),
          pltpu.VMEM((CHUNK, H), jnp.float32),
          pltpu.VMEM((seq_len, H), jnp.float32),
          pltpu.VMEM((1, H), jnp.float32),
          pltpu.VMEM((H,), jnp.float32),
          pltpu.VMEM((H,), jnp.float32),
          pltpu.SemaphoreType.DMA,
          pltpu.SemaphoreType.DMA,
          pltpu.SemaphoreType.DMA,
          pltpu.SemaphoreType.DMA,
          pltpu.SemaphoreType.DMA,
          pltpu.SemaphoreType.DMA,
      ],
  )
  def call(ids_hbm, word_hbm, pos_hbm, tok_hbm, gam_hbm, bet_hbm, out_hbm,
           idx0, idx1, buf0, buf1, obuf0, obuf1, addv, tokv, gamv, betv,
           sem0, sem1, osem0, osem1, isem0, isem1):
    wid = lax.axis_index("s") * nc + lax.axis_index("c")
    base = wid * rows_per_w

    pltpu.sync_copy(pos_hbm.at[pl.ds(0, seq_len)], addv)
    pltpu.sync_copy(tok_hbm.at[pl.ds(0, 1)], tokv)
    pltpu.sync_copy(gam_hbm, gamv)
    pltpu.sync_copy(bet_hbm, betv)

    def add_tok(i, _):
      for j in range(NJ):
        sl = pl.ds(j * LANES, LANES)
        addv[i, sl] = addv[i, sl] + tokv[0, sl]
      return 0
    lax.fori_loop(0, seq_len, add_tok, 0)

    g_regs = [gamv[pl.ds(j * LANES, LANES)] for j in range(NJ)]
    b_regs = [betv[pl.ds(j * LANES, LANES)] for j in range(NJ)]

    def compute_ln(buf, obuf, chunk_id):
      row0 = chunk_id * CHUNK

      @plsc.parallel_loop(0, CHUNK, unroll=4)
      def row(r):
        p = lax.rem(row0 + r, seq_len)
        t = []
        for j in range(NJ):
          sl = pl.ds(j * LANES, LANES)
          t.append(buf[r, sl] + addv[p, sl])
        s = t[0] + t[1]
        q = t[0] * t[0] + t[1] * t[1]
        for j in range(2, NJ):
          s = s + t[j]
          q = q + t[j] * t[j]
        mean = jnp.sum(s) * jnp.float32(1.0 / H)
        msq = jnp.sum(q) * jnp.float32(1.0 / H)
        var = msq - mean * mean
        x = var + jnp.float32(EPS)
        i32 = lax.bitcast_convert_type(x, jnp.int32)
        i32 = jnp.int32(0x5F3759DF) - lax.shift_right_logical(i32, 1)
        y = lax.bitcast_convert_type(i32, jnp.float32)
        for _ in range(2):
          y = y * (jnp.float32(1.5) - jnp.float32(0.5) * x * y * y)
        c = mean * y
        for j in range(NJ):
          obuf[r, pl.ds(j * LANES, LANES)] = (
              (t[j] * y - c) * g_regs[j] + b_regs[j])

    def idx_src(g):
      return ids_hbm.at[pl.ds(base + g * CHUNK, CHUNK)]

    def out_dst(g):
      return out_hbm.at[pl.ds(base + g * CHUNK, CHUNK)]

    pltpu.async_copy(idx_src(0), idx0, isem0)
    pltpu.async_copy(idx_src(1), idx1, isem1)
    pltpu.make_async_copy(idx_src(0), idx0, isem0).wait()
    pltpu.async_copy(word_hbm.at[idx0], buf0, sem0)

    def body(t, _):
      g0 = 2 * t
      g1 = g0 + 1
      not_last = t < n_chunks // 2 - 1
      not_first = t > 0

      pltpu.make_async_copy(word_hbm.at[idx0], buf0, sem0).wait()

      @pl.when(not_last)
      def _():
        pltpu.async_copy(idx_src(g0 + 2), idx0, isem0)

      pltpu.make_async_copy(idx_src(g1), idx1, isem1).wait()
      pltpu.async_copy(word_hbm.at[idx1], buf1, sem1)

      @pl.when(not_first)
      def _():
        pltpu.make_async_copy(obuf0, out_dst(g0 - 2), osem0).wait()

      compute_ln(buf0, obuf0, g0)
      pltpu.async_copy(obuf0, out_dst(g0), osem0)

      pltpu.make_async_copy(word_hbm.at[idx1], buf1, sem1).wait()

      @pl.when(not_last)
      def _():
        pltpu.async_copy(idx_src(g1 + 2), idx1, isem1)
        pltpu.make_async_copy(idx_src(g0 + 2), idx0, isem0).wait()
        pltpu.async_copy(word_hbm.at[idx0], buf0, sem0)

      @pl.when(not_first)
      def _():
        pltpu.make_async_copy(obuf1, out_dst(g1 - 2), osem1).wait()

      compute_ln(buf1, obuf1, g1)
      pltpu.async_copy(obuf1, out_dst(g1), osem1)
      return 0

    lax.fori_loop(0, n_chunks // 2, body, 0)

    pltpu.make_async_copy(obuf0, out_dst(n_chunks - 2), osem0).wait()
    pltpu.make_async_copy(obuf1, out_dst(n_chunks - 1), osem1).wait()

  return call


def kernel(input_ids, entity_ids, word_emb, pos_emb, tok_emb, ent_emb,
           ln_gamma, ln_beta):
  del entity_ids, ent_emb
  bsz, seq_len = input_ids.shape
  n_rows = bsz * seq_len
  ids = input_ids.reshape(n_rows).astype(jnp.int32)
  call = _make_sc_call(n_rows, seq_len)
  out = call(ids, word_emb, pos_emb, tok_emb, ln_gamma, ln_beta)
  return out.reshape(bsz, seq_len, H)
